# baseline (speedup 1.0000x reference)
import functools

import jax
import jax.numpy as jnp
from jax import lax
from jax.experimental import pallas as pl
from jax.experimental.pallas import tpu as pltpu
from jax.experimental.pallas import tpu_sc as plsc

_NC = 2
_NS = 16
_NW = _NC * _NS
_ICHUNK = 128


def _dist_argmin_body(x_ref, w_ref, xsq_ref, esq_ref, idx_ref):
    s = lax.dot_general(
        x_ref[0], w_ref[...],
        (((0,), (0,)), ((), ())),
        preferred_element_type=jnp.float32,
    )
    dist = (xsq_ref[0] - 2.0 * s) + esq_ref[...]
    idx_ref[0, 0, :] = jnp.argmin(dist, axis=1).astype(jnp.int32)


def _argmin_indices(x3, weight, x_sq, e_sq):
    b, d, m = x3.shape
    k = weight.shape[1]
    idx3 = pl.pallas_call(
        _dist_argmin_body,
        grid=(b,),
        in_specs=[
            pl.BlockSpec((1, d, m), lambda i: (i, 0, 0)),
            pl.BlockSpec((d, k), lambda i: (0, 0)),
            pl.BlockSpec((1, m, 1), lambda i: (i, 0, 0)),
            pl.BlockSpec((1, k), lambda i: (0, 0)),
        ],
        out_specs=pl.BlockSpec((1, 1, m), lambda i: (i, 0, 0)),
        out_shape=jax.ShapeDtypeStruct((b, 1, m), jnp.int32),
    )(x3, weight, x_sq, e_sq)
    return idx3.reshape(b * m)


def _sc_gather(table, idx, n, d):
    bpw = n // _NW
    nchunk = bpw // _ICHUNK
    idx3 = idx.reshape(_NW, nchunk, _ICHUNK)
    mesh = plsc.VectorSubcoreMesh(core_axis_name="c", subcore_axis_name="s")

    @functools.partial(
        pl.kernel,
        mesh=mesh,
        out_type=jax.ShapeDtypeStruct((_NW, nchunk, _ICHUNK, d), jnp.float32),
        scratch_types=[
            pltpu.VMEM((nchunk, _ICHUNK), jnp.int32),
            pltpu.VMEM((nchunk, _ICHUNK, d), jnp.float32),
            pltpu.SemaphoreType.DMA,
        ],
        compiler_params=pltpu.CompilerParams(use_tc_tiling_on_sc=False),
    )
    def gather_kernel(table_hbm, idx_hbm, out_hbm, idx_v, rows_v, sem):
        wid = lax.axis_index("s") * _NC + lax.axis_index("c")
        pltpu.sync_copy(idx_hbm.at[wid], idx_v)
        copies = [
            pltpu.async_copy(table_hbm.at[idx_v.at[j]], rows_v.at[j], sem)
            for j in range(nchunk)
        ]
        for c in copies:
            c.wait()
        pltpu.sync_copy(rows_v, out_hbm.at[wid])

    return gather_kernel(table, idx3).reshape(n, d)


def kernel(x, weight):
    b, d, h, w = x.shape
    k = weight.shape[1]
    n = b * h * w
    x_flat = jnp.moveaxis(x, 1, -1).reshape(-1, d)
    emb_t = weight.T
    x_sq = jnp.sum(x_flat * x_flat, axis=1, keepdims=True)
    e_sq = jnp.sum(emb_t * emb_t, axis=1)[None, :]
    x3 = x.reshape(b, d, h * w)
    idx = _argmin_indices(x3, weight, x_sq.reshape(b, h * w, 1), e_sq)
    result_flat = _sc_gather(emb_t, idx, n, d)
    result = jnp.moveaxis(result_flat.reshape(b, h, w, d), -1, 1)
    return result, idx.reshape(b, h, w)


# speedup vs baseline: 1.1343x; 1.1343x over previous
import functools

import jax
import jax.numpy as jnp
from jax import lax
from jax.experimental import pallas as pl
from jax.experimental.pallas import tpu as pltpu
from jax.experimental.pallas import tpu_sc as plsc

_NC = 2
_NS = 16
_NW = _NC * _NS
_ICHUNK = 128


def _fused_body(x_ref, w_ref, res_ref, idx_ref):
    x = x_ref[0]
    w = w_ref[...]
    xsq = jnp.sum(x * x, axis=0)[:, None]
    esq = jnp.sum(w * w, axis=0)[None, :]
    s = lax.dot_general(
        x, w, (((0,), (0,)), ((), ())), preferred_element_type=jnp.float32,
    )
    dist = (xsq - 2.0 * s) + esq
    idx = jnp.argmin(dist, axis=1).astype(jnp.int32)
    idx_ref[0, 0, :] = idx
    k = w.shape[1]
    onehot = (idx[:, None] == lax.broadcasted_iota(jnp.int32, (1, k), 1))
    res_ref[0] = lax.dot_general(
        w, onehot.astype(jnp.float32),
        (((1,), (1,)), ((), ())), preferred_element_type=jnp.float32,
        precision=lax.Precision.HIGHEST,
    )


def _fused_nearest(x3, weight):
    b, d, m = x3.shape
    k = weight.shape[1]
    res, idx3 = pl.pallas_call(
        _fused_body,
        grid=(b,),
        in_specs=[
            pl.BlockSpec((1, d, m), lambda i: (i, 0, 0)),
            pl.BlockSpec((d, k), lambda i: (0, 0)),
        ],
        out_specs=[
            pl.BlockSpec((1, d, m), lambda i: (i, 0, 0)),
            pl.BlockSpec((1, 1, m), lambda i: (i, 0, 0)),
        ],
        out_shape=[
            jax.ShapeDtypeStruct((b, d, m), jnp.float32),
            jax.ShapeDtypeStruct((b, 1, m), jnp.int32),
        ],
    )(x3, weight)
    return res, idx3


def _sc_gather(table, idx, n, d):
    bpw = n // _NW
    nchunk = bpw // _ICHUNK
    idx3 = idx.reshape(_NW, nchunk, _ICHUNK)
    mesh = plsc.VectorSubcoreMesh(core_axis_name="c", subcore_axis_name="s")

    @functools.partial(
        pl.kernel,
        mesh=mesh,
        out_type=jax.ShapeDtypeStruct((_NW, nchunk, _ICHUNK, d), jnp.float32),
        scratch_types=[
            pltpu.VMEM((nchunk, _ICHUNK), jnp.int32),
            pltpu.VMEM((nchunk, _ICHUNK, d), jnp.float32),
            pltpu.SemaphoreType.DMA,
        ],
        compiler_params=pltpu.CompilerParams(use_tc_tiling_on_sc=False),
    )
    def gather_kernel(table_hbm, idx_hbm, out_hbm, idx_v, rows_v, sem):
        wid = lax.axis_index("s") * _NC + lax.axis_index("c")
        pltpu.sync_copy(idx_hbm.at[wid], idx_v)
        copies = [
            pltpu.async_copy(table_hbm.at[idx_v.at[j]], rows_v.at[j], sem)
            for j in range(nchunk)
        ]
        for c in copies:
            c.wait()
        pltpu.sync_copy(rows_v, out_hbm.at[wid])

    return gather_kernel(table, idx3).reshape(n, d)


def kernel(x, weight):
    b, d, h, w = x.shape
    res, idx3 = _fused_nearest(x.reshape(b, d, h * w), weight)
    return res.reshape(b, d, h, w), idx3.reshape(b, h, w)


# speedup vs baseline: 1.9601x; 1.7281x over previous
import functools

import jax
import jax.numpy as jnp
from jax import lax
from jax.experimental import pallas as pl
from jax.experimental.pallas import tpu as pltpu
from jax.experimental.pallas import tpu_sc as plsc

_NC = 2
_NS = 16
_NW = _NC * _NS
_ICHUNK = 128


def _fused_body(x_ref, w_ref, res_ref, idx_ref):
    x = x_ref[0]
    w = w_ref[...]
    k = w.shape[1]
    xsq = jnp.sum(x * x, axis=0)[None, :]
    esq = jnp.sum(w * w, axis=0)[:, None]
    s = lax.dot_general(
        w, x, (((0,), (0,)), ((), ())), preferred_element_type=jnp.float32,
    )
    dist = (xsq - 2.0 * s) + esq
    idx = jnp.argmin(dist, axis=0).astype(jnp.int32)
    idx_ref[0, 0, :] = idx
    onehot = (
        lax.broadcasted_iota(jnp.int32, (k, 1), 0) == idx[None, :]
    ).astype(jnp.bfloat16)
    w_hi = w.astype(jnp.bfloat16)
    r1 = w - w_hi.astype(jnp.float32)
    w_mid = r1.astype(jnp.bfloat16)
    w_lo = (r1 - w_mid.astype(jnp.float32)).astype(jnp.bfloat16)
    dims = (((1,), (0,)), ((), ()))
    res = lax.dot_general(
        w_hi, onehot, dims, preferred_element_type=jnp.float32,
    )
    res += lax.dot_general(
        w_mid, onehot, dims, preferred_element_type=jnp.float32,
    )
    res += lax.dot_general(
        w_lo, onehot, dims, preferred_element_type=jnp.float32,
    )
    res_ref[0] = res


def _fused_nearest(x3, weight):
    b, d, m = x3.shape
    k = weight.shape[1]
    res, idx3 = pl.pallas_call(
        _fused_body,
        grid=(b,),
        in_specs=[
            pl.BlockSpec((1, d, m), lambda i: (i, 0, 0)),
            pl.BlockSpec((d, k), lambda i: (0, 0)),
        ],
        out_specs=[
            pl.BlockSpec((1, d, m), lambda i: (i, 0, 0)),
            pl.BlockSpec((1, 1, m), lambda i: (i, 0, 0)),
        ],
        out_shape=[
            jax.ShapeDtypeStruct((b, d, m), jnp.float32),
            jax.ShapeDtypeStruct((b, 1, m), jnp.int32),
        ],
    )(x3, weight)
    return res, idx3


def _sc_gather(table, idx, n, d):
    bpw = n // _NW
    nchunk = bpw // _ICHUNK
    idx3 = idx.reshape(_NW, nchunk, _ICHUNK)
    mesh = plsc.VectorSubcoreMesh(core_axis_name="c", subcore_axis_name="s")

    @functools.partial(
        pl.kernel,
        mesh=mesh,
        out_type=jax.ShapeDtypeStruct((_NW, nchunk, _ICHUNK, d), jnp.float32),
        scratch_types=[
            pltpu.VMEM((nchunk, _ICHUNK), jnp.int32),
            pltpu.VMEM((nchunk, _ICHUNK, d), jnp.float32),
            pltpu.SemaphoreType.DMA,
        ],
        compiler_params=pltpu.CompilerParams(use_tc_tiling_on_sc=False),
    )
    def gather_kernel(table_hbm, idx_hbm, out_hbm, idx_v, rows_v, sem):
        wid = lax.axis_index("s") * _NC + lax.axis_index("c")
        pltpu.sync_copy(idx_hbm.at[wid], idx_v)
        copies = [
            pltpu.async_copy(table_hbm.at[idx_v.at[j]], rows_v.at[j], sem)
            for j in range(nchunk)
        ]
        for c in copies:
            c.wait()
        pltpu.sync_copy(rows_v, out_hbm.at[wid])

    return gather_kernel(table, idx3).reshape(n, d)


def kernel(x, weight):
    b, d, h, w = x.shape
    res, idx3 = _fused_nearest(x.reshape(b, d, h * w), weight)
    return res.reshape(b, d, h, w), idx3.reshape(b, h, w)


# speedup vs baseline: 2.4895x; 1.2701x over previous
import functools

import jax
import jax.numpy as jnp
from jax import lax
from jax.experimental import pallas as pl
from jax.experimental.pallas import tpu as pltpu
from jax.experimental.pallas import tpu_sc as plsc

_NC = 2
_NS = 16
_NW = _NC * _NS
_ICHUNK = 128


def _fused_body(x_ref, w_ref, res_ref, idx_ref):
    w = w_ref[...]
    d, k = w.shape
    esq = jnp.sum(w * w, axis=0)[:, None]
    w_hi = w.astype(jnp.bfloat16)
    r1 = w - w_hi.astype(jnp.float32)
    w_mid = r1.astype(jnp.bfloat16)
    w_lo = (r1 - w_mid.astype(jnp.float32)).astype(jnp.bfloat16)
    w3 = jnp.concatenate([w_hi, w_mid, w_lo], axis=0)
    for j in range(x_ref.shape[0]):
        x = x_ref[j]
        xsq = jnp.sum(x * x, axis=0)[None, :]
        s = lax.dot_general(
            w, x, (((0,), (0,)), ((), ())),
            preferred_element_type=jnp.float32,
        )
        dist = (xsq - 2.0 * s) + esq
        idx = jnp.argmin(dist, axis=0).astype(jnp.int32)
        idx_ref[j, 0, :] = idx
        onehot = (
            lax.broadcasted_iota(jnp.int32, (k, 1), 0) == idx[None, :]
        ).astype(jnp.bfloat16)
        res3 = lax.dot_general(
            w3, onehot, (((1,), (0,)), ((), ())),
            preferred_element_type=jnp.float32,
        )
        res_ref[j] = (res3[:d] + res3[d:2 * d]) + res3[2 * d:]


def _fused_nearest(x3, weight):
    b, d, m = x3.shape
    k = weight.shape[1]
    bpb = 4 if b % 4 == 0 else 1
    res, idx3 = pl.pallas_call(
        _fused_body,
        grid=(b // bpb,),
        in_specs=[
            pl.BlockSpec((bpb, d, m), lambda i: (i, 0, 0)),
            pl.BlockSpec((d, k), lambda i: (0, 0)),
        ],
        out_specs=[
            pl.BlockSpec((bpb, d, m), lambda i: (i, 0, 0)),
            pl.BlockSpec((bpb, 1, m), lambda i: (i, 0, 0)),
        ],
        out_shape=[
            jax.ShapeDtypeStruct((b, d, m), jnp.float32),
            jax.ShapeDtypeStruct((b, 1, m), jnp.int32),
        ],
    )(x3, weight)
    return res, idx3


def _sc_gather(table, idx, n, d):
    bpw = n // _NW
    nchunk = bpw // _ICHUNK
    idx3 = idx.reshape(_NW, nchunk, _ICHUNK)
    mesh = plsc.VectorSubcoreMesh(core_axis_name="c", subcore_axis_name="s")

    @functools.partial(
        pl.kernel,
        mesh=mesh,
        out_type=jax.ShapeDtypeStruct((_NW, nchunk, _ICHUNK, d), jnp.float32),
        scratch_types=[
            pltpu.VMEM((nchunk, _ICHUNK), jnp.int32),
            pltpu.VMEM((nchunk, _ICHUNK, d), jnp.float32),
            pltpu.SemaphoreType.DMA,
        ],
        compiler_params=pltpu.CompilerParams(use_tc_tiling_on_sc=False),
    )
    def gather_kernel(table_hbm, idx_hbm, out_hbm, idx_v, rows_v, sem):
        wid = lax.axis_index("s") * _NC + lax.axis_index("c")
        pltpu.sync_copy(idx_hbm.at[wid], idx_v)
        copies = [
            pltpu.async_copy(table_hbm.at[idx_v.at[j]], rows_v.at[j], sem)
            for j in range(nchunk)
        ]
        for c in copies:
            c.wait()
        pltpu.sync_copy(rows_v, out_hbm.at[wid])

    return gather_kernel(table, idx3).reshape(n, d)


def kernel(x, weight):
    b, d, h, w = x.shape
    res, idx3 = _fused_nearest(x.reshape(b, d, h * w), weight)
    return res.reshape(b, d, h, w), idx3.reshape(b, h, w)


# speedup vs baseline: 2.5487x; 1.0238x over previous
import functools

import jax
import jax.numpy as jnp
from jax import lax
from jax.experimental import pallas as pl
from jax.experimental.pallas import tpu as pltpu
from jax.experimental.pallas import tpu_sc as plsc

_NC = 2
_NS = 16
_NW = _NC * _NS
_ICHUNK = 128


def _fused_body(x_ref, w_ref, res_ref, idx_ref):
    w = w_ref[...]
    d, k = w.shape
    esq = jnp.sum(w * w, axis=0)[:, None]
    w_hi = w.astype(jnp.bfloat16)
    r1 = w - w_hi.astype(jnp.float32)
    w_mid = r1.astype(jnp.bfloat16)
    w_lo = (r1 - w_mid.astype(jnp.float32)).astype(jnp.bfloat16)
    w3 = jnp.concatenate([w_hi, w_mid, w_lo], axis=0)
    ss = [
        lax.dot_general(
            w, x_ref[j], (((0,), (0,)), ((), ())),
            preferred_element_type=jnp.float32,
        )
        for j in range(x_ref.shape[0])
    ]
    for j in range(x_ref.shape[0]):
        x = x_ref[j]
        xsq = jnp.sum(x * x, axis=0)[None, :]
        s = ss[j]
        dist = (xsq - 2.0 * s) + esq
        idx = jnp.argmin(dist, axis=0).astype(jnp.int32)
        idx_ref[j, 0, :] = idx
        onehot = (
            lax.broadcasted_iota(jnp.int32, (k, 1), 0) == idx[None, :]
        ).astype(jnp.bfloat16)
        res3 = lax.dot_general(
            w3, onehot, (((1,), (0,)), ((), ())),
            preferred_element_type=jnp.float32,
        )
        res_ref[j] = (res3[:d] + res3[d:2 * d]) + res3[2 * d:]


def _fused_nearest(x3, weight):
    b, d, m = x3.shape
    k = weight.shape[1]
    bpb = 4 if b % 4 == 0 else 1
    res, idx3 = pl.pallas_call(
        _fused_body,
        grid=(b // bpb,),
        in_specs=[
            pl.BlockSpec((bpb, d, m), lambda i: (i, 0, 0)),
            pl.BlockSpec((d, k), lambda i: (0, 0)),
        ],
        out_specs=[
            pl.BlockSpec((bpb, d, m), lambda i: (i, 0, 0)),
            pl.BlockSpec((bpb, 1, m), lambda i: (i, 0, 0)),
        ],
        out_shape=[
            jax.ShapeDtypeStruct((b, d, m), jnp.float32),
            jax.ShapeDtypeStruct((b, 1, m), jnp.int32),
        ],
    )(x3, weight)
    return res, idx3


def _sc_gather(table, idx, n, d):
    bpw = n // _NW
    nchunk = bpw // _ICHUNK
    idx3 = idx.reshape(_NW, nchunk, _ICHUNK)
    mesh = plsc.VectorSubcoreMesh(core_axis_name="c", subcore_axis_name="s")

    @functools.partial(
        pl.kernel,
        mesh=mesh,
        out_type=jax.ShapeDtypeStruct((_NW, nchunk, _ICHUNK, d), jnp.float32),
        scratch_types=[
            pltpu.VMEM((nchunk, _ICHUNK), jnp.int32),
            pltpu.VMEM((nchunk, _ICHUNK, d), jnp.float32),
            pltpu.SemaphoreType.DMA,
        ],
        compiler_params=pltpu.CompilerParams(use_tc_tiling_on_sc=False),
    )
    def gather_kernel(table_hbm, idx_hbm, out_hbm, idx_v, rows_v, sem):
        wid = lax.axis_index("s") * _NC + lax.axis_index("c")
        pltpu.sync_copy(idx_hbm.at[wid], idx_v)
        copies = [
            pltpu.async_copy(table_hbm.at[idx_v.at[j]], rows_v.at[j], sem)
            for j in range(nchunk)
        ]
        for c in copies:
            c.wait()
        pltpu.sync_copy(rows_v, out_hbm.at[wid])

    return gather_kernel(table, idx3).reshape(n, d)


def kernel(x, weight):
    b, d, h, w = x.shape
    res, idx3 = _fused_nearest(x.reshape(b, d, h * w), weight)
    return res.reshape(b, d, h, w), idx3.reshape(b, h, w)
